# baseline (speedup 1.0000x reference)
import functools

import jax
import jax.numpy as jnp
from jax import lax
from jax.experimental import pallas as pl
from jax.experimental.pallas import tpu as pltpu
from jax.experimental.pallas import tpu_sc as plsc

_INFO = plsc.get_sparse_core_info()
_NC = _INFO.num_cores
_NS = _INFO.num_subcores
_NW = _NC * _NS
_CHUNKS = (32, 96, 128, 128, 128)
_N_HBM = 1


@jax.jit
def _embed_lookup(labels, table):
    (b,) = labels.shape
    v, d = table.shape
    b_per_w = b // _NW
    assert sum(_CHUNKS) == b_per_w
    offs = [sum(_CHUNKS[:j]) for j in range(len(_CHUNKS))]

    mesh = plsc.VectorSubcoreMesh(core_axis_name="c", subcore_axis_name="s")

    @functools.partial(
        pl.kernel,
        mesh=mesh,
        out_type=jax.ShapeDtypeStruct((b, d), jnp.float32),
        scratch_types=[
            pltpu.VMEM((b_per_w,), jnp.int32),
            pltpu.VMEM((b_per_w, d), jnp.float32),
            pltpu.VMEM_SHARED((v, d), jnp.float32),
        ]
        + [pltpu.SemaphoreType.DMA] * len(_CHUNKS)
        + [pltpu.SemaphoreType.DMA],
    )
    def run(labels_hbm, table_hbm, out_hbm, idx_v, rows_v, table_sh, *sems):
        gather_sems, store_sem = sems[: len(_CHUNKS)], sems[len(_CHUNKS)]
        sid = lax.axis_index("s")
        wid = sid * _NC + lax.axis_index("c")
        base = wid * b_per_w
        pltpu.sync_copy(labels_hbm.at[pl.ds(base, b_per_w)], idx_v)
        gathers = [
            pltpu.async_copy(
                table_hbm.at[idx_v.at[pl.ds(offs[j], _CHUNKS[j])]],
                rows_v.at[pl.ds(offs[j], _CHUNKS[j])],
                gather_sems[j],
            )
            for j in range(_N_HBM)
        ]
        @pl.when(sid == 0)
        def _():
            pltpu.sync_copy(table_hbm, table_sh)

        plsc.subcore_barrier()
        gathers += [
            pltpu.async_copy(
                table_sh.at[idx_v.at[pl.ds(offs[j], _CHUNKS[j])]],
                rows_v.at[pl.ds(offs[j], _CHUNKS[j])],
                gather_sems[j],
            )
            for j in range(_N_HBM, len(_CHUNKS))
        ]
        stores = []
        for j in range(len(_CHUNKS)):
            gathers[j].wait()
            stores.append(
                pltpu.async_copy(
                    rows_v.at[pl.ds(offs[j], _CHUNKS[j])],
                    out_hbm.at[pl.ds(base + offs[j], _CHUNKS[j])],
                    store_sem,
                )
            )
        for s in stores:
            s.wait()

    return run(labels, table)


def kernel(labels, table):
    return _embed_lookup(labels.astype(jnp.int32), table)


# speedup vs baseline: 1.0195x; 1.0195x over previous
import functools

import jax
import jax.numpy as jnp
from jax import lax
from jax.experimental import pallas as pl
from jax.experimental.pallas import tpu as pltpu
from jax.experimental.pallas import tpu_sc as plsc

_INFO = plsc.get_sparse_core_info()
_NC = _INFO.num_cores
_NS = _INFO.num_subcores
_NW = _NC * _NS
_CHUNK = 128


@jax.jit
def _embed_lookup(labels2d, table):
    n_rows, chunk = labels2d.shape
    v, d = table.shape
    b = n_rows * chunk
    b_per_w = b // _NW
    nch = b_per_w // chunk

    mesh = plsc.VectorSubcoreMesh(core_axis_name="c", subcore_axis_name="s")

    @functools.partial(
        pl.kernel,
        mesh=mesh,
        out_type=jax.ShapeDtypeStruct((b, d), jnp.float32),
        scratch_types=[
            pltpu.VMEM((nch, chunk), jnp.int32),
            pltpu.VMEM((b_per_w, d), jnp.float32),
            pltpu.VMEM_SHARED((v, d), jnp.float32),
        ]
        + [pltpu.SemaphoreType.DMA] * nch
        + [pltpu.SemaphoreType.DMA],
    )
    def run(labels_hbm, table_hbm, out_hbm, idx_v, rows_v, table_sh, *sems):
        gather_sems, lbl_sem = sems[:nch], sems[nch]
        sid = lax.axis_index("s")
        wid = sid * _NC + lax.axis_index("c")
        base = wid * b_per_w
        lbl = pltpu.async_copy(
            labels_hbm.at[pl.ds(wid * nch, nch)], idx_v, lbl_sem
        )
        @pl.when(sid == 0)
        def _():
            pltpu.sync_copy(table_hbm, table_sh)

        plsc.subcore_barrier()
        lbl.wait()
        gathers = [
            pltpu.async_copy(
                table_sh.at[idx_v.at[j]],
                rows_v.at[pl.ds(j * chunk, chunk)],
                gather_sems[j],
            )
            for j in range(nch)
        ]
        stores = []
        for j in range(nch):
            gathers[j].wait()
            stores.append(
                pltpu.async_copy(
                    rows_v.at[pl.ds(j * chunk, chunk)],
                    out_hbm.at[pl.ds(base + j * chunk, chunk)],
                    lbl_sem,
                )
            )
        for s in stores:
            s.wait()

    return run(labels2d, table)


def kernel(labels, table):
    (b,) = labels.shape
    labels2d = labels.astype(jnp.int32).reshape(b // _CHUNK, _CHUNK)
    return _embed_lookup(labels2d, table)
